# Initial kernel scaffold; baseline (speedup 1.0000x reference)
import jax
import jax.numpy as jnp
from jax.experimental import pallas as pl


def kernel(x, w_xy, w_xh, w_hy, kh, ky):
    raise NotImplementedError("write your pallas kernel here")


# speedup vs baseline: 161.2708x; 161.2708x over previous
import functools

import jax
import jax.numpy as jnp
from jax.experimental import pallas as pl
from jax.experimental.pallas import tpu as pltpu


def _kwta_block(s, kf, iters_val, iters_idx):
    R, N = s.shape
    lo = jnp.min(s, axis=1, keepdims=True)
    hi = jnp.max(s, axis=1, keepdims=True) + 1.0

    def body_a(_, c):
        lo, hi = c
        mid = jnp.floor((lo + hi) * 0.5)
        cnt = jnp.sum(jnp.where(s >= mid, 1.0, 0.0), axis=1, keepdims=True)
        ge = cnt >= kf
        return jnp.where(ge, mid, lo), jnp.where(ge, hi, mid)

    lo, hi = jax.lax.fori_loop(0, iters_val, body_a, (lo, hi))
    t = lo
    gt = s > t
    cnt_gt = jnp.sum(jnp.where(gt, 1.0, 0.0), axis=1, keepdims=True)
    r = kf - cnt_gt
    eq = s == t
    idx = jax.lax.broadcasted_iota(jnp.int32, (R, N), 1).astype(jnp.float32)

    lo2 = jnp.full((R, 1), -1.0, jnp.float32)
    hi2 = jnp.full((R, 1), float(N - 1), jnp.float32)

    def body_b(_, c):
        lo2, hi2 = c
        mid = jnp.floor((lo2 + hi2) * 0.5)
        cnt = jnp.sum(jnp.where(eq & (idx <= mid), 1.0, 0.0), axis=1,
                      keepdims=True)
        ge = cnt >= r
        return jnp.where(ge, lo2, mid), jnp.where(ge, mid, hi2)

    lo2, hi2 = jax.lax.fori_loop(0, iters_idx, body_b, (lo2, hi2))
    m = hi2
    return jnp.where(gt | (eq & (idx <= m)), 1.0, 0.0)


def _wta_body(ks_ref, x_ref, wxy_ref, wxh_ref, why_ref, h_ref, y_ref,
              *, iters_val_h, iters_idx_h, iters_val_y, iters_idx_y):
    x = x_ref[...]
    kh = ks_ref[0].astype(jnp.float32)
    ky = ks_ref[1].astype(jnp.float32)
    y0 = jnp.dot(x, wxy_ref[...], preferred_element_type=jnp.float32)
    s_h = jnp.dot(x, wxh_ref[...], preferred_element_type=jnp.float32)
    h = _kwta_block(s_h, kh, iters_val_h, iters_idx_h)
    h_ref[...] = h
    inh = jnp.dot(h.astype(jnp.bfloat16), why_ref[...],
                  preferred_element_type=jnp.float32)
    y = _kwta_block(y0 - inh, ky, iters_val_y, iters_idx_y)
    y_ref[...] = y


def _ceil_log2(n):
    k = 0
    while (1 << k) < n:
        k += 1
    return k


def kernel(x, w_xy, w_xh, w_hy, kh, ky):
    B, NX = x.shape
    NY = w_xy.shape[1]
    NH = w_xh.shape[1]
    RB = 128

    xb = x.astype(jnp.bfloat16)
    wxy = w_xy.astype(jnp.bfloat16)
    wxh = w_xh.astype(jnp.bfloat16)
    why = w_hy.astype(jnp.bfloat16)
    ks = jnp.stack([jnp.asarray(kh, jnp.int32), jnp.asarray(ky, jnp.int32)])

    iters_val_h = _ceil_log2(NX + 2)
    iters_idx_h = _ceil_log2(NH)
    iters_val_y = _ceil_log2(NX + NH + 2)
    iters_idx_y = _ceil_log2(NY)

    body = functools.partial(
        _wta_body,
        iters_val_h=iters_val_h, iters_idx_h=iters_idx_h,
        iters_val_y=iters_val_y, iters_idx_y=iters_idx_y,
    )

    h, y = pl.pallas_call(
        body,
        grid_spec=pltpu.PrefetchScalarGridSpec(
            num_scalar_prefetch=1,
            grid=(B // RB,),
            in_specs=[
                pl.BlockSpec((RB, NX), lambda i, ks: (i, 0)),
                pl.BlockSpec((NX, NY), lambda i, ks: (0, 0)),
                pl.BlockSpec((NX, NH), lambda i, ks: (0, 0)),
                pl.BlockSpec((NH, NY), lambda i, ks: (0, 0)),
            ],
            out_specs=[
                pl.BlockSpec((RB, NH), lambda i, ks: (i, 0)),
                pl.BlockSpec((RB, NY), lambda i, ks: (i, 0)),
            ],
        ),
        out_shape=[
            jax.ShapeDtypeStruct((B, NH), jnp.float32),
            jax.ShapeDtypeStruct((B, NY), jnp.float32),
        ],
        compiler_params=pltpu.CompilerParams(
            dimension_semantics=("arbitrary",),
        ),
    )(ks, xb, wxy, wxh, why)
    return h, y


# speedup vs baseline: 242.6585x; 1.5047x over previous
import functools

import jax
import jax.numpy as jnp
import numpy as np
from jax.experimental import pallas as pl
from jax.experimental.pallas import tpu as pltpu


@functools.lru_cache(maxsize=None)
def _prefix_mats(n):
    gs = n // 128
    j = np.arange(n)[:, None]
    g = np.arange(128)[None, :]
    mg = ((j // gs) <= g).astype(np.float32)
    l2 = ((j % gs) <= g).astype(np.float32)
    return mg, l2


def _kwta_block(s, kf, mg, l2):
    R, N = s.shape
    gs = N // 128

    lo = jnp.min(s, axis=1, keepdims=True)
    hi = jnp.max(s, axis=1, keepdims=True) + 1.0

    def cond_a(c):
        lo, hi = c
        return jnp.max(hi - lo) > 1.0

    def body_a(c):
        lo, hi = c
        mid = jnp.floor((lo + hi) * 0.5)
        cnt = jnp.sum(jnp.where(s >= mid, 1.0, 0.0), axis=1, keepdims=True)
        ge = cnt >= kf
        return jnp.where(ge, mid, lo), jnp.where(ge, hi, mid)

    lo, hi = jax.lax.while_loop(cond_a, body_a, (lo, hi))
    t = lo
    gt = s > t
    cnt_gt = jnp.sum(jnp.where(gt, 1.0, 0.0), axis=1, keepdims=True)
    r = kf - cnt_gt
    eq = s == t

    eqf = jnp.where(eq, 1.0, 0.0).astype(jnp.bfloat16)
    p = jnp.dot(eqf, mg, preferred_element_type=jnp.float32)
    gstar = jnp.sum(jnp.where(p < r, 1.0, 0.0), axis=1, keepdims=True)
    gcol = jax.lax.broadcasted_iota(jnp.int32, (R, 128), 1).astype(jnp.float32)
    before = jnp.sum(jnp.where(gcol == gstar - 1.0, p, 0.0), axis=1,
                     keepdims=True)
    r_in = r - before

    idx = jax.lax.broadcasted_iota(jnp.int32, (R, N), 1).astype(jnp.float32)
    gidx = jnp.floor(idx * (1.0 / gs))
    eqg = jnp.where(eq & (gidx == gstar), 1.0, 0.0).astype(jnp.bfloat16)
    q = jnp.dot(eqg, l2, preferred_element_type=jnp.float32)
    in_range = gcol < float(gs)
    ostar = jnp.sum(jnp.where(in_range & (q < r_in), 1.0, 0.0), axis=1,
                    keepdims=True)
    m = gstar * float(gs) + ostar
    return jnp.where(gt | (eq & (idx <= m)), 1.0, 0.0)


def _wta_body(ks_ref, x_ref, wxy_ref, wxh_ref, why_ref,
              mgh_ref, l2h_ref, mgy_ref, l2y_ref, h_ref, y_ref):
    x = x_ref[...]
    kh = ks_ref[0].astype(jnp.float32)
    ky = ks_ref[1].astype(jnp.float32)
    y0 = jnp.dot(x, wxy_ref[...], preferred_element_type=jnp.float32)
    s_h = jnp.dot(x, wxh_ref[...], preferred_element_type=jnp.float32)
    h = _kwta_block(s_h, kh, mgh_ref[...], l2h_ref[...])
    h_ref[...] = h
    inh = jnp.dot(h.astype(jnp.bfloat16), why_ref[...],
                  preferred_element_type=jnp.float32)
    y = _kwta_block(y0 - inh, ky, mgy_ref[...], l2y_ref[...])
    y_ref[...] = y


def kernel(x, w_xy, w_xh, w_hy, kh, ky):
    B, NX = x.shape
    NY = w_xy.shape[1]
    NH = w_xh.shape[1]
    RB = 128

    xb = x.astype(jnp.bfloat16)
    wxy = w_xy.astype(jnp.bfloat16)
    wxh = w_xh.astype(jnp.bfloat16)
    why = w_hy.astype(jnp.bfloat16)
    ks = jnp.stack([jnp.asarray(kh, jnp.int32), jnp.asarray(ky, jnp.int32)])

    mgh_np, l2h_np = _prefix_mats(NH)
    mgy_np, l2y_np = _prefix_mats(NY)
    mgh = jnp.asarray(mgh_np, jnp.bfloat16)
    l2h = jnp.asarray(l2h_np, jnp.bfloat16)
    mgy = jnp.asarray(mgy_np, jnp.bfloat16)
    l2y = jnp.asarray(l2y_np, jnp.bfloat16)

    full = lambda i, ks: (0, 0)
    rows = lambda i, ks: (i, 0)

    h, y = pl.pallas_call(
        _wta_body,
        grid_spec=pltpu.PrefetchScalarGridSpec(
            num_scalar_prefetch=1,
            grid=(B // RB,),
            in_specs=[
                pl.BlockSpec((RB, NX), rows),
                pl.BlockSpec((NX, NY), full),
                pl.BlockSpec((NX, NH), full),
                pl.BlockSpec((NH, NY), full),
                pl.BlockSpec((NH, 128), full),
                pl.BlockSpec((NH, 128), full),
                pl.BlockSpec((NY, 128), full),
                pl.BlockSpec((NY, 128), full),
            ],
            out_specs=[
                pl.BlockSpec((RB, NH), rows),
                pl.BlockSpec((RB, NY), rows),
            ],
        ),
        out_shape=[
            jax.ShapeDtypeStruct((B, NH), jnp.float32),
            jax.ShapeDtypeStruct((B, NY), jnp.float32),
        ],
        compiler_params=pltpu.CompilerParams(
            dimension_semantics=("arbitrary",),
        ),
    )(ks, xb, wxy, wxh, why, mgh, l2h, mgy, l2y)
    return h, y
